# baseline (speedup 1.0000x reference)
import jax
import jax.numpy as jnp
from jax import lax
from jax.experimental import pallas as pl
from jax.experimental.pallas import tpu as pltpu
from jax.experimental.pallas import tpu_sc as plsc

VOCAB = 100000
D = 768
B = 4
S = 2048
N = B * S

NC, NS = 2, 16
NW = NC * NS
PER_W = N // NW
W_PER_B = S // PER_W
CHUNK = 64
SCHED = (16, 64, 64, 64, 32, 16)
OFFS = tuple(sum(SCHED[:i]) for i in range(len(SCHED)))
NCHUNK = len(SCHED)
NBUF = 2


def _gather_body(ids_hbm, mask_hbm, pos_hbm, lab_hbm, table_hbm,
                 out_hbm, omask_hbm, opos_hbm, olab_hbm,
                 idx_v, *bufs_and_sems):
    rows = bufs_and_sems[:NBUF]
    gsem = bufs_and_sems[NBUF:2 * NBUF]
    ssem = bufs_and_sems[2 * NBUF:3 * NBUF]
    psem = bufs_and_sems[3 * NBUF]

    wid = lax.axis_index("s") * NC + lax.axis_index("c")
    b = wid // W_PER_B
    col = (wid % W_PER_B) * PER_W

    p0 = pltpu.async_copy(mask_hbm.at[b, pl.ds(col, PER_W)],
                          omask_hbm.at[b, pl.ds(col, PER_W)], psem)
    p1 = pltpu.async_copy(pos_hbm.at[b, pl.ds(col, PER_W)],
                          opos_hbm.at[b, pl.ds(col, PER_W)], psem)
    p2 = pltpu.async_copy(lab_hbm.at[b, pl.ds(col, PER_W)],
                          olab_hbm.at[b, pl.ds(col, PER_W)], psem)

    pltpu.sync_copy(ids_hbm.at[b, pl.ds(col, PER_W)], idx_v)

    gathers = [None] * NBUF
    scatters = [None] * NBUF
    for j in range(NCHUNK):
        r = j % NBUF
        if scatters[r] is not None:
            scatters[r].wait()
        sz = SCHED[j]
        gathers[r] = pltpu.async_copy(
            table_hbm.at[idx_v.at[pl.ds(OFFS[j], sz)]],
            rows[r].at[pl.ds(0, sz)], gsem[r])
        if j > 0:
            p = (j - 1) % NBUF
            gathers[p].wait()
            scatters[p] = pltpu.async_copy(
                rows[p].at[pl.ds(0, SCHED[j - 1])],
                out_hbm.at[b, pl.ds(col + OFFS[j - 1], SCHED[j - 1])],
                ssem[p])
    last = (NCHUNK - 1) % NBUF
    gathers[last].wait()
    scatters[last] = pltpu.async_copy(
        rows[last].at[pl.ds(0, SCHED[NCHUNK - 1])],
        out_hbm.at[b, pl.ds(col + OFFS[NCHUNK - 1], SCHED[NCHUNK - 1])],
        ssem[last])
    for j in range(max(0, NCHUNK - NBUF + 1), NCHUNK):
        scatters[j % NBUF].wait()
    p0.wait()
    p1.wait()
    p2.wait()


@jax.jit
def _embed_lookup(input_ids, attention_mask, position_ids, labels,
                  embedding_table):
    mesh = plsc.VectorSubcoreMesh(core_axis_name="c", subcore_axis_name="s")
    k = pl.kernel(
        _gather_body,
        out_type=(
            jax.ShapeDtypeStruct((B, S, D), jnp.float32),
            jax.ShapeDtypeStruct((B, S), jnp.int32),
            jax.ShapeDtypeStruct((B, S), jnp.int32),
            jax.ShapeDtypeStruct((B, S), jnp.int32),
        ),
        mesh=mesh,
        scratch_types=(
            [pltpu.VMEM((PER_W,), jnp.int32)]
            + [pltpu.VMEM((CHUNK, D), jnp.float32) for _ in range(NBUF)]
            + [pltpu.SemaphoreType.DMA] * (2 * NBUF + 1)
        ),
    )
    return k(input_ids, attention_mask, position_ids, labels,
             embedding_table)


def kernel(input_ids, attention_mask, position_ids, labels, embedding_table):
    return _embed_lookup(input_ids, attention_mask, position_ids, labels,
                         embedding_table)


# speedup vs baseline: 1.0067x; 1.0067x over previous
import jax
import jax.numpy as jnp
from jax import lax
from jax.experimental import pallas as pl
from jax.experimental.pallas import tpu as pltpu
from jax.experimental.pallas import tpu_sc as plsc

VOCAB = 100000
D = 768
B = 4
S = 2048
N = B * S

NC, NS = 2, 16
NW = NC * NS
PER_W = N // NW
W_PER_B = S // PER_W
CHUNK = 64
SCHED = (64, 64, 64, 64)
OFFS = tuple(sum(SCHED[:i]) for i in range(len(SCHED)))
NCHUNK = len(SCHED)
NBUF = 2


def _gather_body(ids_hbm, mask_hbm, pos_hbm, lab_hbm, table_hbm,
                 out_hbm, omask_hbm, opos_hbm, olab_hbm,
                 idx_v, *bufs_and_sems):
    rows = bufs_and_sems[:NBUF]
    gsem = bufs_and_sems[NBUF:2 * NBUF]
    ssem = bufs_and_sems[2 * NBUF:3 * NBUF]
    psem = bufs_and_sems[3 * NBUF]

    wid = lax.axis_index("s") * NC + lax.axis_index("c")
    b = wid // W_PER_B
    col = (wid % W_PER_B) * PER_W

    p0 = pltpu.async_copy(mask_hbm.at[b, pl.ds(col, PER_W)],
                          omask_hbm.at[b, pl.ds(col, PER_W)], psem)
    p1 = pltpu.async_copy(pos_hbm.at[b, pl.ds(col, PER_W)],
                          opos_hbm.at[b, pl.ds(col, PER_W)], psem)
    p2 = pltpu.async_copy(lab_hbm.at[b, pl.ds(col, PER_W)],
                          olab_hbm.at[b, pl.ds(col, PER_W)], psem)

    pltpu.sync_copy(ids_hbm.at[b, pl.ds(col, PER_W)], idx_v)

    gathers = [None] * NBUF
    scatters = [None] * NBUF
    for j in range(NCHUNK):
        r = j % NBUF
        if scatters[r] is not None:
            scatters[r].wait()
        sz = SCHED[j]
        gathers[r] = pltpu.async_copy(
            table_hbm.at[idx_v.at[pl.ds(OFFS[j], sz)]],
            rows[r].at[pl.ds(0, sz)], gsem[r])
        if j > 0:
            p = (j - 1) % NBUF
            gathers[p].wait()
            scatters[p] = pltpu.async_copy(
                rows[p].at[pl.ds(0, SCHED[j - 1])],
                out_hbm.at[b, pl.ds(col + OFFS[j - 1], SCHED[j - 1])],
                ssem[p])
    last = (NCHUNK - 1) % NBUF
    gathers[last].wait()
    scatters[last] = pltpu.async_copy(
        rows[last].at[pl.ds(0, SCHED[NCHUNK - 1])],
        out_hbm.at[b, pl.ds(col + OFFS[NCHUNK - 1], SCHED[NCHUNK - 1])],
        ssem[last])
    for j in range(max(0, NCHUNK - NBUF + 1), NCHUNK):
        scatters[j % NBUF].wait()
    p0.wait()
    p1.wait()
    p2.wait()


@jax.jit
def _embed_lookup(input_ids, attention_mask, position_ids, labels,
                  embedding_table):
    mesh = plsc.VectorSubcoreMesh(core_axis_name="c", subcore_axis_name="s")
    k = pl.kernel(
        _gather_body,
        out_type=(
            jax.ShapeDtypeStruct((B, S, D), jnp.float32),
            jax.ShapeDtypeStruct((B, S), jnp.int32),
            jax.ShapeDtypeStruct((B, S), jnp.int32),
            jax.ShapeDtypeStruct((B, S), jnp.int32),
        ),
        mesh=mesh,
        scratch_types=(
            [pltpu.VMEM((PER_W,), jnp.int32)]
            + [pltpu.VMEM((CHUNK, D), jnp.float32) for _ in range(NBUF)]
            + [pltpu.SemaphoreType.DMA] * (2 * NBUF + 1)
        ),
    )
    return k(input_ids, attention_mask, position_ids, labels,
             embedding_table)


def kernel(input_ids, attention_mask, position_ids, labels, embedding_table):
    return _embed_lookup(input_ids, attention_mask, position_ids, labels,
                         embedding_table)


# speedup vs baseline: 1.0122x; 1.0055x over previous
import jax
import jax.numpy as jnp
from jax import lax
from jax.experimental import pallas as pl
from jax.experimental.pallas import tpu as pltpu
from jax.experimental.pallas import tpu_sc as plsc

VOCAB = 100000
D = 768
B = 4
S = 2048
N = B * S

NC, NS = 2, 16
NW = NC * NS
PER_W = N // NW
W_PER_B = S // PER_W
CHUNK = 64
NCHUNK = PER_W // CHUNK
NBUF = 2


def _gather_body(ids_hbm, mask_hbm, pos_hbm, lab_hbm, table_hbm,
                 out_hbm, omask_hbm, opos_hbm, olab_hbm,
                 idx_v, rows0, rows1, gsem, ssem, psem):
    rows = (rows0, rows1)

    wid = lax.axis_index("s") * NC + lax.axis_index("c")
    b = wid // W_PER_B
    col = (wid % W_PER_B) * PER_W

    pairs = ((mask_hbm, omask_hbm), (pos_hbm, opos_hbm), (lab_hbm, olab_hbm))
    arr = wid // B
    row = wid % B

    @pl.when(wid < 3 * B)
    def _start_passthrough():
        for a, (src, dst) in enumerate(pairs):
            @pl.when(arr == a)
            def _():
                pltpu.async_copy(src.at[row], dst.at[row], psem)

    pltpu.sync_copy(ids_hbm.at[b, pl.ds(col, PER_W)], idx_v)

    gathers = [None] * NBUF
    scatters = [None] * NBUF
    for j in range(NCHUNK):
        r = j % NBUF
        if scatters[r] is not None:
            scatters[r].wait()
        gathers[r] = pltpu.async_copy(
            table_hbm.at[idx_v.at[pl.ds(j * CHUNK, CHUNK)]], rows[r],
            gsem)
        if j > 0:
            p = (j - 1) % NBUF
            gathers[p].wait()
            scatters[p] = pltpu.async_copy(
                rows[p], out_hbm.at[b, pl.ds(col + (j - 1) * CHUNK, CHUNK)],
                ssem)
    last = (NCHUNK - 1) % NBUF
    gathers[last].wait()
    scatters[last] = pltpu.async_copy(
        rows[last], out_hbm.at[b, pl.ds(col + (NCHUNK - 1) * CHUNK, CHUNK)],
        ssem)
    for j in range(max(0, NCHUNK - NBUF + 1), NCHUNK):
        scatters[j % NBUF].wait()

    @pl.when(wid < 3 * B)
    def _drain_passthrough():
        pltpu.make_async_copy(mask_hbm.at[0], omask_hbm.at[0], psem).wait()


@jax.jit
def _embed_lookup(input_ids, attention_mask, position_ids, labels,
                  embedding_table):
    mesh = plsc.VectorSubcoreMesh(core_axis_name="c", subcore_axis_name="s")
    k = pl.kernel(
        _gather_body,
        out_type=(
            jax.ShapeDtypeStruct((B, S, D), jnp.float32),
            jax.ShapeDtypeStruct((B, S), jnp.int32),
            jax.ShapeDtypeStruct((B, S), jnp.int32),
            jax.ShapeDtypeStruct((B, S), jnp.int32),
        ),
        mesh=mesh,
        scratch_types=(
            [pltpu.VMEM((PER_W,), jnp.int32)]
            + [pltpu.VMEM((CHUNK, D), jnp.float32) for _ in range(NBUF)]
            + [pltpu.SemaphoreType.DMA] * 3
        ),
    )
    return k(input_ids, attention_mask, position_ids, labels,
             embedding_table)


def kernel(input_ids, attention_mask, position_ids, labels, embedding_table):
    return _embed_lookup(input_ids, attention_mask, position_ids, labels,
                         embedding_table)
